# baseline (speedup 1.0000x reference)
import functools

import jax
import jax.numpy as jnp
from jax import lax
from jax.experimental import pallas as pl
from jax.experimental.pallas import tpu as pltpu
from jax.experimental.pallas import tpu_sc as plsc

N_CONT = 13
N_CAT = 26
VOCAB = 100000
EDIM = 16
BATCH = 16384
N_IN = N_CONT + N_CAT
N_EMB = N_CAT * EDIM
N_OUT = N_EMB + N_CONT
_SH2 = 10

_INFO = plsc.get_sparse_core_info()
_NC = _INFO.num_cores
_NS = _INFO.num_subcores
_L = _INFO.num_lanes
_NW = _NC * _NS

_TOTAL = BATCH * N_CAT
_PER_W = _TOTAL // _NW
_BPW = BATCH // _NW
_CB = 256
_IC = _CB * N_CAT
_NCHUNK = _BPW // _CB
_GRP = _IC // _L

_CB1 = 512
_CB3 = 512


def _make_indices(x_in):
  def body(x_ref, idx1_ref, idx2_ref):
    offs1 = lax.broadcasted_iota(jnp.int32, (_CB1, EDIM), 1) * VOCAB
    idx1_ref[...] = x_ref[:, N_CONT:N_CONT + EDIM].astype(jnp.int32) + offs1
    idx2_ref[...] = (
        x_ref[:, N_CONT + _SH2:N_CONT + _SH2 + EDIM].astype(jnp.int32)
        + offs1 + _SH2 * VOCAB)

  return pl.pallas_call(
      body,
      grid=(BATCH // _CB1,),
      in_specs=[pl.BlockSpec((_CB1, N_IN), lambda g: (g, 0))],
      out_specs=[
          pl.BlockSpec((_CB1, EDIM), lambda g: (g, 0)),
          pl.BlockSpec((_CB1, EDIM), lambda g: (g, 0)),
      ],
      out_shape=[
          jax.ShapeDtypeStruct((BATCH, EDIM), jnp.int32),
          jax.ShapeDtypeStruct((BATCH, EDIM), jnp.int32),
      ],
  )(x_in)


def _gather_rows(idx1, idx2, table_flat):
  mesh = plsc.VectorSubcoreMesh(core_axis_name="c", subcore_axis_name="s")

  @functools.partial(
      pl.kernel,
      mesh=mesh,
      out_type=jax.ShapeDtypeStruct((_TOTAL, EDIM), jnp.float32),
      compiler_params=pltpu.CompilerParams(
          use_tc_tiling_on_sc=False, needs_layout_passes=False),
      scratch_types=[
          pltpu.VMEM((_CB, EDIM), jnp.int32),
          pltpu.VMEM((_CB, EDIM), jnp.int32),
          pltpu.VMEM((_IC,), jnp.int32),
          pltpu.VMEM((_IC, EDIM), jnp.float32),
          pltpu.SemaphoreType.DMA,
      ],
  )
  def k(idx1_hbm, idx2_hbm, table_hbm, out_hbm, i1buf, i2buf, idxbuf, rowbuf,
        sem):
    wid = lax.axis_index("s") * _NC + lax.axis_index("c")
    lane = lax.iota(jnp.int32, _L)

    def chunk_body(c, carry):
      b0 = pl.multiple_of(wid * _BPW + c * _CB, _CB)
      pltpu.sync_copy(idx1_hbm.at[pl.ds(b0, _CB)], i1buf)
      pltpu.sync_copy(idx2_hbm.at[pl.ds(b0, _CB)], i2buf)

      def grp_body(g, carry2):
        off = pl.multiple_of(g * _L, _L)
        p = off + lane
        b = lax.div(p, N_CAT)
        i = lax.rem(p, N_CAT)
        lo = i < EDIM
        v1 = plsc.load_gather(i1buf, [b, i], mask=lo)
        v2 = plsc.load_gather(i2buf, [b, i - _SH2], mask=jnp.logical_not(lo))
        idxbuf[pl.ds(off, _L)] = jnp.where(lo, v1, v2)
        return carry2

      lax.fori_loop(0, _GRP, grp_body, 0, unroll=4)

      pltpu.async_copy(table_hbm.at[idxbuf], rowbuf, sem).wait()
      pltpu.sync_copy(rowbuf, out_hbm.at[pl.ds(b0 * N_CAT, _IC)])
      return carry

    lax.fori_loop(0, _NCHUNK, chunk_body, 0)

  return k(idx1, idx2, table_flat)


def _assemble(emb_rows, x_in):
  def body(emb_ref, x_ref, out_ref):
    e3 = emb_ref[...].reshape(_CB3, N_CAT, EDIM)
    for i in range(N_CAT):
      out_ref[:, i * EDIM:(i + 1) * EDIM] = e3[:, i, :]
    out_ref[:, N_EMB:] = x_ref[:, :N_CONT]

  return pl.pallas_call(
      body,
      grid=(BATCH // _CB3,),
      in_specs=[
          pl.BlockSpec((_CB3 * N_CAT, EDIM), lambda g: (g, 0)),
          pl.BlockSpec((_CB3, N_IN), lambda g: (g, 0)),
      ],
      out_specs=pl.BlockSpec((_CB3, N_OUT), lambda g: (g, 0)),
      out_shape=jax.ShapeDtypeStruct((BATCH, N_OUT), jnp.float32),
  )(emb_rows, x_in)


def kernel(x_in, tables):
  table_flat = tables.reshape(N_CAT * VOCAB, EDIM)
  idx1, idx2 = _make_indices(x_in)
  emb_rows = _gather_rows(idx1, idx2, table_flat)
  return _assemble(emb_rows, x_in)


# speedup vs baseline: 1.1637x; 1.1637x over previous
import functools

import jax
import jax.numpy as jnp
from jax import lax
from jax.experimental import pallas as pl
from jax.experimental.pallas import tpu as pltpu
from jax.experimental.pallas import tpu_sc as plsc

N_CONT = 13
N_CAT = 26
VOCAB = 100000
EDIM = 16
BATCH = 16384
N_IN = N_CONT + N_CAT
N_EMB = N_CAT * EDIM
N_OUT = N_EMB + N_CONT
N_PAD = 512
_SH2 = 10

_INFO = plsc.get_sparse_core_info()
_NC = _INFO.num_cores
_NS = _INFO.num_subcores
_L = _INFO.num_lanes
_NW = _NC * _NS

_BPW = BATCH // _NW
_CB = 128
_IC = _CB * N_CAT
_NCHUNK = _BPW // _CB
_GRP = _IC // _L

_CB1 = 512
_CB3 = 512


def _make_slabs(x_in):
  def body(x_ref, idx1_ref, idx2_ref, cont_ref):
    offs1 = lax.broadcasted_iota(jnp.int32, (_CB1, EDIM), 1) * VOCAB
    idx1_ref[...] = x_ref[:, N_CONT:N_CONT + EDIM].astype(jnp.int32) + offs1
    idx2_ref[...] = (
        x_ref[:, N_CONT + _SH2:N_CONT + _SH2 + EDIM].astype(jnp.int32)
        + offs1 + _SH2 * VOCAB)
    cont_ref[...] = x_ref[:, :EDIM]

  return pl.pallas_call(
      body,
      grid=(BATCH // _CB1,),
      in_specs=[pl.BlockSpec((_CB1, N_IN), lambda g: (g, 0))],
      out_specs=[
          pl.BlockSpec((_CB1, EDIM), lambda g: (g, 0)),
          pl.BlockSpec((_CB1, EDIM), lambda g: (g, 0)),
          pl.BlockSpec((_CB1, EDIM), lambda g: (g, 0)),
      ],
      out_shape=[
          jax.ShapeDtypeStruct((BATCH, EDIM), jnp.int32),
          jax.ShapeDtypeStruct((BATCH, EDIM), jnp.int32),
          jax.ShapeDtypeStruct((BATCH, EDIM), jnp.float32),
      ],
  )(x_in)


def _gather_assemble(idx1, idx2, cont, table_flat):
  mesh = plsc.VectorSubcoreMesh(core_axis_name="c", subcore_axis_name="s")

  @functools.partial(
      pl.kernel,
      mesh=mesh,
      out_type=jax.ShapeDtypeStruct((BATCH * N_PAD,), jnp.float32),
      compiler_params=pltpu.CompilerParams(
          use_tc_tiling_on_sc=False, needs_layout_passes=False),
      scratch_types=[
          pltpu.VMEM((_CB, EDIM), jnp.int32),
          pltpu.VMEM((_CB, EDIM), jnp.int32),
          pltpu.VMEM((_CB, EDIM), jnp.float32),
          pltpu.VMEM((_IC,), jnp.int32),
          pltpu.VMEM((_IC, EDIM), jnp.float32),
          pltpu.VMEM((_CB * N_PAD,), jnp.float32),
          pltpu.SemaphoreType.DMA,
      ],
  )
  def k(idx1_hbm, idx2_hbm, cont_hbm, table_hbm, out_hbm,
        i1buf, i2buf, cbuf, idxbuf, rowbuf, outbuf, sem):
    wid = lax.axis_index("s") * _NC + lax.axis_index("c")
    lane = lax.iota(jnp.int32, _L)

    def chunk_body(c, carry):
      b0 = pl.multiple_of(wid * _BPW + c * _CB, _CB)
      pltpu.sync_copy(idx1_hbm.at[pl.ds(b0, _CB)], i1buf)
      pltpu.sync_copy(idx2_hbm.at[pl.ds(b0, _CB)], i2buf)
      pltpu.sync_copy(cont_hbm.at[pl.ds(b0, _CB)], cbuf)

      def grp_body(g, carry2):
        off = pl.multiple_of(g * _L, _L)
        p = off + lane
        b = lax.div(p, N_CAT)
        i = lax.rem(p, N_CAT)
        lo = i < EDIM
        v1 = plsc.load_gather(i1buf, [b, i], mask=lo)
        v2 = plsc.load_gather(i2buf, [b, i - _SH2], mask=jnp.logical_not(lo))
        idxbuf[pl.ds(off, _L)] = jnp.where(lo, v1, v2)
        return carry2

      lax.fori_loop(0, _GRP, grp_body, 0, unroll=4)

      pltpu.async_copy(table_hbm.at[idxbuf], rowbuf, sem).wait()

      def row_body(b, carry3):
        o = b * N_PAD
        for i in range(N_CAT):
          outbuf[pl.ds(o + i * EDIM, _L)] = rowbuf[b * N_CAT + i]
        outbuf[pl.ds(o + N_EMB, _L)] = cbuf[b]
        return carry3

      lax.fori_loop(0, _CB, row_body, 0, unroll=2)

      pltpu.sync_copy(outbuf, out_hbm.at[pl.ds(b0 * N_PAD, _CB * N_PAD)])
      return carry

    lax.fori_loop(0, _NCHUNK, chunk_body, 0)

  return k(idx1, idx2, cont, table_flat)


def _compact(out_pad):
  def body(pad_ref, out_ref):
    e2 = pad_ref[...].reshape(_CB3, N_PAD)
    out_ref[...] = e2[:, :N_OUT]

  return pl.pallas_call(
      body,
      grid=(BATCH // _CB3,),
      in_specs=[pl.BlockSpec((_CB3 * N_PAD,), lambda g: (g,))],
      out_specs=pl.BlockSpec((_CB3, N_OUT), lambda g: (g, 0)),
      out_shape=jax.ShapeDtypeStruct((BATCH, N_OUT), jnp.float32),
  )(out_pad)


def kernel(x_in, tables):
  table_flat = tables.reshape(N_CAT * VOCAB, EDIM)
  idx1, idx2, cont = _make_slabs(x_in)
  out_pad = _gather_assemble(idx1, idx2, cont, table_flat)
  return _compact(out_pad)


# speedup vs baseline: 1.1983x; 1.0297x over previous
import functools

import jax
import jax.numpy as jnp
from jax import lax
from jax.experimental import pallas as pl
from jax.experimental.pallas import tpu as pltpu
from jax.experimental.pallas import tpu_sc as plsc

N_CONT = 13
N_CAT = 26
VOCAB = 100000
EDIM = 16
BATCH = 16384

_INFO = plsc.get_sparse_core_info()
_NC = _INFO.num_cores
_NS = _INFO.num_subcores
_L = _INFO.num_lanes
_NW = _NC * _NS

_TOTAL_ROWS = BATCH * N_CAT
_PER_W = _TOTAL_ROWS // _NW
_CHUNK = 3328
_NCHUNK = _PER_W // _CHUNK
_VECS = _CHUNK // _L


def _emb_gather(x_cat_flat, table_flat):
  mesh = plsc.VectorSubcoreMesh(core_axis_name="c", subcore_axis_name="s")

  @functools.partial(
      pl.kernel,
      mesh=mesh,
      out_type=jax.ShapeDtypeStruct((_TOTAL_ROWS, EDIM), jnp.float32),
      compiler_params=pltpu.CompilerParams(use_tc_tiling_on_sc=False),
      scratch_types=[
          pltpu.VMEM((_CHUNK,), jnp.float32),
          pltpu.VMEM((_CHUNK,), jnp.int32),
          pltpu.VMEM((_CHUNK, EDIM), jnp.float32),
          pltpu.SemaphoreType.DMA,
      ],
  )
  def k(xcat_hbm, table_hbm, out_hbm, xbuf, idxbuf, rowbuf, sem):
    wid = lax.axis_index("s") * _NC + lax.axis_index("c")
    base_w = wid * _PER_W
    lane = lax.iota(jnp.int32, _L)

    def chunk_body(c, carry):
      base = pl.multiple_of(base_w + c * _CHUNK, 8)
      pltpu.sync_copy(xcat_hbm.at[pl.ds(base, _CHUNK)], xbuf)

      def vec_body(j, carry2):
        off = pl.multiple_of(j * _L, _L)
        v = xbuf[pl.ds(off, _L)].astype(jnp.int32)
        pos = base + off + lane
        idxbuf[pl.ds(off, _L)] = v + lax.rem(pos, N_CAT) * VOCAB
        return carry2

      lax.fori_loop(0, _VECS, vec_body, 0, unroll=4)

      pltpu.async_copy(table_hbm.at[idxbuf], rowbuf, sem).wait()
      pltpu.sync_copy(rowbuf, out_hbm.at[pl.ds(base, _CHUNK)])
      return carry

    lax.fori_loop(0, _NCHUNK, chunk_body, 0)

  return k(x_cat_flat, table_flat)


def kernel(x_in, tables):
  x_cat_flat = x_in[:, N_CONT:].reshape(-1)
  table_flat = tables.reshape(N_CAT * VOCAB, EDIM)
  emb = _emb_gather(x_cat_flat, table_flat)
  x = emb.reshape(BATCH, N_CAT * EDIM)
  return jnp.concatenate([x, x_in[:, :N_CONT]], axis=1)
